# Initial kernel scaffold; baseline (speedup 1.0000x reference)
import jax
import jax.numpy as jnp
from jax.experimental import pallas as pl


def kernel(x, target):
    raise NotImplementedError("write your pallas kernel here")


# speedup vs baseline: 2.8472x; 2.8472x over previous
import functools
import math

import jax
import jax.numpy as jnp
from jax import lax
from jax.experimental import pallas as pl
from jax.experimental.pallas import tpu as pltpu
from jax.experimental.pallas import tpu_sc as plsc

_SIZE = 32000
_PAD = 0
_SMOOTH = 0.1
_FILL = _SMOOTH / (_SIZE - 2)
_CONF = 1.0 - _SMOOTH
_ENT_C = _CONF * math.log(_CONF) + (_SIZE - 2) * _FILL * math.log(_FILL)

_N_ROWS = 2048
_BR = 256
_BC = 6400
_GR = _N_ROWS // _BR
_GC = _SIZE // _BC

_NC = 2
_NS = 16
_NW = _NC * _NS
_RPW = _N_ROWS // _NW
_L = 16


def _tc_body(tgt_ref, x_ref, out_ref):
    i = pl.program_id(0)
    j = pl.program_id(1)

    @pl.when((i == 0) & (j == 0))
    def _init():
        out_ref[0, 0] = 0.0

    valid = tgt_ref[...] != _PAD
    xb = x_ref[...]
    acc = -_FILL * jnp.sum(jnp.where(valid, xb, 0.0))

    @pl.when(j == 0)
    def _first_col_block():
        col0 = jnp.where(valid, xb[:, 0:1], 0.0)
        nv = jnp.sum(jnp.where(valid, 1.0, 0.0))
        out_ref[0, 0] += _FILL * jnp.sum(col0) + _ENT_C * nv

    out_ref[0, 0] += acc


def _sc_gather_body(xf_hbm, tgt_hbm, out_hbm, tgt_v, idx_v, vals_v, acc_v, sem):
    wid = lax.axis_index("s") * _NC + lax.axis_index("c")
    base = wid * _RPW
    pltpu.sync_copy(tgt_hbm.at[pl.ds(base, _RPW)], tgt_v)
    for k in range(_RPW // _L):
        t16 = tgt_v[pl.ds(k * _L, _L)]
        i16 = base + k * _L + lax.iota(jnp.int32, _L)
        idx_v[pl.ds(k * _L, _L)] = i16 * _SIZE + t16
    pltpu.async_copy(xf_hbm.at[idx_v], vals_v, sem).wait()
    acc = jnp.zeros((_L,), jnp.float32)
    for k in range(_RPW // _L):
        t16 = tgt_v[pl.ds(k * _L, _L)]
        v16 = vals_v[pl.ds(k * _L, _L)]
        acc = acc + jnp.where(t16 != _PAD, v16, 0.0)
    acc_v[...] = acc
    pltpu.sync_copy(acc_v, out_hbm.at[pl.ds(wid * _L, _L)])


@functools.lru_cache(maxsize=1)
def _sc_gather():
    return pl.kernel(
        _sc_gather_body,
        out_type=jax.ShapeDtypeStruct((_NW * _L,), jnp.float32),
        mesh=plsc.VectorSubcoreMesh(
            core_axis_name="c", subcore_axis_name="s",
            num_cores=_NC, num_subcores=_NS),
        scratch_types=[
            pltpu.VMEM((_RPW,), jnp.int32),
            pltpu.VMEM((_RPW,), jnp.int32),
            pltpu.VMEM((_RPW,), jnp.float32),
            pltpu.VMEM((_L,), jnp.float32),
            pltpu.SemaphoreType.DMA,
        ],
    )


def kernel(x, target):
    tgt2 = target.reshape(_N_ROWS, 1)
    tc_out = pl.pallas_call(
        _tc_body,
        grid=(_GR, _GC),
        in_specs=[
            pl.BlockSpec((_BR, 1), lambda i, j: (i, 0)),
            pl.BlockSpec((_BR, _BC), lambda i, j: (i, j)),
        ],
        out_specs=pl.BlockSpec((1, 1), lambda i, j: (0, 0),
                               memory_space=pltpu.SMEM),
        out_shape=jax.ShapeDtypeStruct((1, 1), jnp.float32),
    )(tgt2, x)
    xf = x.reshape(_N_ROWS * _SIZE)
    sc_part = jnp.sum(_sc_gather()(xf, target))
    return tc_out[0, 0] + (_FILL - _CONF) * sc_part


# speedup vs baseline: 3.0819x; 1.0824x over previous
import functools
import math

import jax
import jax.numpy as jnp
from jax import lax
from jax.experimental import pallas as pl
from jax.experimental.pallas import tpu as pltpu
from jax.experimental.pallas import tpu_sc as plsc

_SIZE = 32000
_PAD = 0
_SMOOTH = 0.1
_FILL = _SMOOTH / (_SIZE - 2)
_CONF = 1.0 - _SMOOTH
_ENT_C = _CONF * math.log(_CONF) + (_SIZE - 2) * _FILL * math.log(_FILL)

_N_ROWS = 2048
_BR = 64
_GR = _N_ROWS // _BR

_NC = 2
_NS = 16
_NW = _NC * _NS
_RPW = _N_ROWS // _NW
_L = 16


def _tc_body(tgt_ref, x_ref, out_ref):
    i = pl.program_id(0)

    @pl.when(i == 0)
    def _init():
        out_ref[0, 0] = 0.0

    valid = tgt_ref[...] != _PAD
    xb = x_ref[...]
    rs = jnp.sum(xb, axis=1, keepdims=True)
    col0 = jnp.where(valid, xb[:, 0:1], 0.0)
    nv = jnp.sum(jnp.where(valid, 1.0, 0.0))
    out_ref[0, 0] += (-_FILL * jnp.sum(jnp.where(valid, rs, 0.0))
                      + _FILL * jnp.sum(col0) + _ENT_C * nv)


def _sc_gather_body(xf_hbm, tgt_hbm, out_hbm, tgt_v, idx_v, vals_v, acc_v, sem):
    wid = lax.axis_index("s") * _NC + lax.axis_index("c")
    base = wid * _RPW
    pltpu.sync_copy(tgt_hbm.at[pl.ds(base, _RPW)], tgt_v)
    for k in range(_RPW // _L):
        t16 = tgt_v[pl.ds(k * _L, _L)]
        i16 = base + k * _L + lax.iota(jnp.int32, _L)
        idx_v[pl.ds(k * _L, _L)] = i16 * _SIZE + t16
    pltpu.async_copy(xf_hbm.at[idx_v], vals_v, sem).wait()
    acc = jnp.zeros((_L,), jnp.float32)
    for k in range(_RPW // _L):
        t16 = tgt_v[pl.ds(k * _L, _L)]
        v16 = vals_v[pl.ds(k * _L, _L)]
        acc = acc + jnp.where(t16 != _PAD, v16, 0.0)
    acc_v[...] = acc
    pltpu.sync_copy(acc_v, out_hbm.at[pl.ds(wid * _L, _L)])


@functools.lru_cache(maxsize=1)
def _sc_gather():
    return pl.kernel(
        _sc_gather_body,
        out_type=jax.ShapeDtypeStruct((_NW * _L,), jnp.float32),
        mesh=plsc.VectorSubcoreMesh(
            core_axis_name="c", subcore_axis_name="s",
            num_cores=_NC, num_subcores=_NS),
        scratch_types=[
            pltpu.VMEM((_RPW,), jnp.int32),
            pltpu.VMEM((_RPW,), jnp.int32),
            pltpu.VMEM((_RPW,), jnp.float32),
            pltpu.VMEM((_L,), jnp.float32),
            pltpu.SemaphoreType.DMA,
        ],
    )


def kernel(x, target):
    tgt2 = target.reshape(_N_ROWS, 1)
    tc_out = pl.pallas_call(
        _tc_body,
        grid=(_GR,),
        in_specs=[
            pl.BlockSpec((_BR, 1), lambda i: (i, 0)),
            pl.BlockSpec((_BR, _SIZE), lambda i: (i, 0)),
        ],
        out_specs=pl.BlockSpec((1, 1), lambda i: (0, 0),
                               memory_space=pltpu.SMEM),
        out_shape=jax.ShapeDtypeStruct((1, 1), jnp.float32),
    )(tgt2, x)
    xf = x.reshape(_N_ROWS * _SIZE)
    sc_part = jnp.sum(_sc_gather()(xf, target))
    return tc_out[0, 0] + (_FILL - _CONF) * sc_part
